# Initial kernel scaffold; baseline (speedup 1.0000x reference)
import jax
import jax.numpy as jnp
from jax.experimental import pallas as pl


def kernel(indices, cis_x, cis_y, cis_z):
    raise NotImplementedError("write your pallas kernel here")


# speedup vs baseline: 1.6790x; 1.6790x over previous
import jax
import jax.numpy as jnp
from jax import lax
from jax.experimental import pallas as pl
from jax.experimental.pallas import tpu as pltpu
from jax.experimental.pallas import tpu_sc as plsc

DX, DY, DZ = 24, 24, 16
DOUT = DX + DY + DZ
NC, NS = 2, 16
NW = NC * NS
CHUNK = 512
L = 16


def _body(ix_hbm, iy_hbm, iz_hbm, tx_hbm, ty_hbm, tz_hbm, out_hbm,
          ix_v, iy_v, iz_v, tx_v, ty_v, tz_v, out_v):
    n = out_hbm.shape[0] // DOUT
    tok_per_w = n // NW
    n_chunks = tok_per_w // CHUNK
    wid = lax.axis_index("s") * NC + lax.axis_index("c")
    base_w = wid * tok_per_w
    pltpu.sync_copy(tx_hbm, tx_v)
    pltpu.sync_copy(ty_hbm, ty_v)
    pltpu.sync_copy(tz_hbm, tz_v)
    lo_mask = jax.lax.iota(jnp.int32, L) < 8

    def group(s, carry):
        ixg = ix_v[pl.ds(s * L, L)] * DX
        iyg = iy_v[pl.ds(s * L, L)] * DY
        izg = iz_v[pl.ds(s * L, L)] * DZ
        for k in range(L):
            a = ixg[k]
            b = iyg[k]
            c = izg[k]
            o = (s * L + k) * DOUT
            out_v[pl.ds(o, L)] = tx_v[pl.ds(a, L)]
            out_v[pl.ds(o + L, L)] = jnp.where(
                lo_mask, tx_v[pl.ds(a + L, L)], ty_v[pl.ds(b, L)])
            out_v[pl.ds(o + 2 * L, L)] = ty_v[pl.ds(b + L, L)]
            out_v[pl.ds(o + 3 * L, L)] = tz_v[pl.ds(c, L)]
        return carry

    @pl.loop(0, n_chunks)
    def chunk_loop(j):
        base = pl.multiple_of(base_w + j * CHUNK, CHUNK)
        pltpu.sync_copy(ix_hbm.at[pl.ds(base, CHUNK)], ix_v)
        pltpu.sync_copy(iy_hbm.at[pl.ds(base, CHUNK)], iy_v)
        pltpu.sync_copy(iz_hbm.at[pl.ds(base, CHUNK)], iz_v)
        lax.fori_loop(0, CHUNK // L, group, 0)
        pltpu.sync_copy(out_v, out_hbm.at[pl.ds(base * DOUT, CHUNK * DOUT)])


@jax.jit
def kernel(indices, cis_x, cis_y, cis_z):
    n = indices.shape[0]
    ix = indices[:, 0]
    iy = indices[:, 1]
    iz = indices[:, 2]
    tx = jnp.pad(cis_x.reshape(-1), (0, 8))
    ty = jnp.pad(cis_y.reshape(-1), (8, 0))
    tz = cis_z.reshape(-1)
    mesh = plsc.VectorSubcoreMesh(core_axis_name="c", subcore_axis_name="s",
                                  num_cores=NC, num_subcores=NS)
    out = pl.kernel(
        _body,
        out_type=jax.ShapeDtypeStruct((n * DOUT,), jnp.float32),
        mesh=mesh,
        scratch_types=[
            pltpu.VMEM((CHUNK,), jnp.int32),
            pltpu.VMEM((CHUNK,), jnp.int32),
            pltpu.VMEM((CHUNK,), jnp.int32),
            pltpu.VMEM((1024 * DX + 8,), jnp.float32),
            pltpu.VMEM((1024 * DY + 8,), jnp.float32),
            pltpu.VMEM((512 * DZ,), jnp.float32),
            pltpu.VMEM((CHUNK * DOUT,), jnp.float32),
        ],
    )(ix, iy, iz, tx, ty, tz)
    return out.reshape(1, n, DOUT // 2, 2)
